# baseline (speedup 1.0000x reference)
import functools

import numpy as np
import jax
import jax.numpy as jnp
from jax import lax
from jax.experimental import pallas as pl
from jax.experimental.pallas import tpu as pltpu
from jax.experimental.pallas import tpu_sc as plsc

N = 10000
E = 320000
D = 128
NB = 10
RN = 64
MAX_R = 3.5
HF = 64
NT = 16
EH = E // 2
EPT = EH // NT
CH = 40
NCH = EPT // CH
RQ = 624
REM = N - NT * RQ
ZR = 156
SIN = float(np.sin(np.pi / 8.0))
COS = float(np.cos(np.pi / 8.0))
INV_SQRT_D = float(1.0 / np.sqrt(D))
INV_SQRT_NB = float(1.0 / np.sqrt(NB))
INV_SQRT_RN = float(1.0 / np.sqrt(RN))
INV_SQRT_NEI = float(1.0 / np.sqrt(32.0))
INV_SQRT_N = float(1.0 / np.sqrt(10000.0))

@functools.cache
def _get_mesh():
    return plsc.VectorSubcoreMesh(core_axis_name="c", subcore_axis_name="s")


_BP = 3200


def _wbar_body(evtA, evtB, f1, f2, o):
    v = jnp.concatenate([evtA[...], evtB[...]], axis=1)
    el2 = jnp.sum(v * v, axis=0, keepdims=True)
    elen = jnp.sqrt(el2)
    step = MAX_R / (NB - 1)
    centers = lax.broadcasted_iota(jnp.int32, (NB, 1), 0).astype(jnp.float32) * step
    diff = (elen - centers) / step
    eln = jnp.exp(-diff * diff) * (float(np.sqrt(NB)) / 1.12)
    u = 2.0 * (elen / MAX_R - 1.0)
    cut = (1.0 - jnp.cos(jnp.pi * u)) * 0.5
    cut = jnp.where(u > 0.0, 0.0, cut)
    cut = jnp.where(u < -1.0, 1.0, cut)
    elnb = eln.astype(jnp.bfloat16)
    ht = jax.nn.silu(
        jnp.dot(f1[...].astype(jnp.bfloat16), elnb,
                preferred_element_type=jnp.float32) * INV_SQRT_NB)
    ht = ht * (cut * INV_SQRT_RN)
    wt = jnp.dot(f2[...].astype(jnp.bfloat16), ht.astype(jnp.bfloat16),
                 preferred_element_type=jnp.float32)
    wb = wt.T
    pA = wb[:_BP]
    pB = wb[_BP:]
    o[0] = jnp.concatenate([pA[:, :HF], pB[:, :HF]], axis=1)
    o[1] = jnp.concatenate([pA[:, HF:], pB[:, HF:]], axis=1)


def _wbar_call(evt, f1t, f2t):
    nblk = EH // _BP
    return pl.pallas_call(
        _wbar_body,
        grid=(nblk,),
        in_specs=[
            pl.BlockSpec((3, _BP), lambda i: (0, i)),
            pl.BlockSpec((3, _BP), lambda i, n=nblk: (0, i + n)),
            pl.BlockSpec((RN, NB), lambda i: (0, 0)),
            pl.BlockSpec((D, RN), lambda i: (0, 0)),
        ],
        out_specs=pl.BlockSpec((2, _BP, D), lambda i: (0, i, 0)),
        out_shape=jax.ShapeDtypeStruct((2, EH, D), jnp.float32),
    )(evt, evt, f1t, f2t)


_BN = 2000


def _node_in_body(x_r, wsc_r, wl1_r, s_o, xa_o, xb_o):
    xv = x_r[...]
    s_o[...] = jnp.dot(xv, wsc_r[...], preferred_element_type=jnp.float32) * INV_SQRT_D
    xl = jnp.dot(xv, wl1_r[...], preferred_element_type=jnp.float32) * INV_SQRT_D
    xa_o[...] = xl[:, :HF]
    xb_o[...] = xl[:, HF:]


def _node_in_call(x, wsc, wl1):
    wspec = pl.BlockSpec((D, D), lambda i: (0, 0))
    return pl.pallas_call(
        _node_in_body,
        grid=(N // _BN,),
        in_specs=[pl.BlockSpec((_BN, D), lambda i: (i, 0)), wspec, wspec],
        out_specs=[pl.BlockSpec((_BN, D), lambda i: (i, 0)),
                   pl.BlockSpec((_BN, HF), lambda i: (i, 0)),
                   pl.BlockSpec((_BN, HF), lambda i: (i, 0))],
        out_shape=[jax.ShapeDtypeStruct((N, D), jnp.float32),
                   jax.ShapeDtypeStruct((N, HF), jnp.float32),
                   jax.ShapeDtypeStruct((N, HF), jnp.float32)],
    )(x, wsc, wl1)


def _epi_mid_body(s_r, agg_r, wl2_r, wsc_r, wl1_r, s_o, xa_o, xb_o):
    a = jnp.concatenate([agg_r[0], agg_r[1]], axis=1)
    out = (jnp.dot(a, wl2_r[...], preferred_element_type=jnp.float32)
           * (INV_SQRT_NEI * INV_SQRT_D))
    h = jax.nn.silu(SIN * s_r[...] + COS * out)
    s_o[...] = jnp.dot(h, wsc_r[...], preferred_element_type=jnp.float32) * INV_SQRT_D
    xl = jnp.dot(h, wl1_r[...], preferred_element_type=jnp.float32) * INV_SQRT_D
    xa_o[...] = xl[:, :HF]
    xb_o[...] = xl[:, HF:]


def _epi_mid_call(s, agg, wl2, wsc, wl1):
    wspec = pl.BlockSpec((D, D), lambda i: (0, 0))
    return pl.pallas_call(
        _epi_mid_body,
        grid=(N // _BN,),
        in_specs=[pl.BlockSpec((_BN, D), lambda i: (i, 0)),
                  pl.BlockSpec((2, _BN, HF), lambda i: (0, i, 0)),
                  wspec, wspec, wspec],
        out_specs=[pl.BlockSpec((_BN, D), lambda i: (i, 0)),
                   pl.BlockSpec((_BN, HF), lambda i: (i, 0)),
                   pl.BlockSpec((_BN, HF), lambda i: (i, 0))],
        out_shape=[jax.ShapeDtypeStruct((N, D), jnp.float32),
                   jax.ShapeDtypeStruct((N, HF), jnp.float32),
                   jax.ShapeDtypeStruct((N, HF), jnp.float32)],
    )(s, agg, wl2, wsc, wl1)


def _epi_last_body(s_r, agg_r, wl2_r, wl1_r, h_o, xa_o, xb_o):
    a = jnp.concatenate([agg_r[0], agg_r[1]], axis=1)
    out = (jnp.dot(a, wl2_r[...], preferred_element_type=jnp.float32)
           * (INV_SQRT_NEI * INV_SQRT_D))
    h = jax.nn.silu(SIN * s_r[...] + COS * out)
    h_o[...] = h
    xl = jnp.dot(h, wl1_r[...], preferred_element_type=jnp.float32) * INV_SQRT_D
    xa_o[...] = xl[:, :HF]
    xb_o[...] = xl[:, HF:]


def _epi_last_call(s, agg, wl2, wl1):
    wspec = pl.BlockSpec((D, D), lambda i: (0, 0))
    return pl.pallas_call(
        _epi_last_body,
        grid=(N // _BN,),
        in_specs=[pl.BlockSpec((_BN, D), lambda i: (i, 0)),
                  pl.BlockSpec((2, _BN, HF), lambda i: (0, i, 0)),
                  wspec, wspec],
        out_specs=[pl.BlockSpec((_BN, D), lambda i: (i, 0)),
                   pl.BlockSpec((_BN, HF), lambda i: (i, 0)),
                   pl.BlockSpec((_BN, HF), lambda i: (i, 0))],
        out_shape=[jax.ShapeDtypeStruct((N, D), jnp.float32),
                   jax.ShapeDtypeStruct((N, HF), jnp.float32),
                   jax.ShapeDtypeStruct((N, HF), jnp.float32)],
    )(s, agg, wl2, wl1)


def _final_body(h_r, xa_r, xb_r, s2_r, wsc_r, wl2_r, o):
    hsum = jnp.sum(h_r[...], axis=0, keepdims=True)
    x0 = jnp.sum(xa_r[...] * s2_r[0], axis=0, keepdims=True)
    x1 = jnp.sum(xb_r[...] * s2_r[1], axis=0, keepdims=True)
    xsum = jnp.concatenate([x0, x1], axis=1)
    ssum = jnp.dot(hsum, wsc_r[...], preferred_element_type=jnp.float32) * INV_SQRT_D
    osum = (jnp.dot(xsum, wl2_r[...], preferred_element_type=jnp.float32)
            * (INV_SQRT_NEI * INV_SQRT_D))
    o[...] = (SIN * ssum + COS * osum) * INV_SQRT_N


def _final_call(h1, xa2, xb2, s2, wsc2, wl22):
    return pl.pallas_call(
        _final_body,
        grid=(1,),
        in_specs=[pl.BlockSpec((N, D), lambda i: (0, 0)),
                  pl.BlockSpec((N, HF), lambda i: (0, 0)),
                  pl.BlockSpec((N, HF), lambda i: (0, 0)),
                  pl.BlockSpec((2, N, HF), lambda i: (0, 0, 0)),
                  pl.BlockSpec((D, 1), lambda i: (0, 0)),
                  pl.BlockSpec((D, 1), lambda i: (0, 0))],
        out_specs=pl.BlockSpec((1, 1), lambda i: (0, 0)),
        out_shape=jax.ShapeDtypeStruct((1, 1), jnp.float32),
    )(h1, xa2, xb2, s2, wsc2, wl22)


def _zero_rows(zbuf):
    def zrow(i, _):
        for j in range(HF // 16):
            zbuf[i, pl.ds(j * 16, 16)] = jnp.zeros((16,), jnp.float32)
        return 0
    lax.fori_loop(0, ZR, zrow, 0)


def _zero_agg(s, aggsh, zbuf):
    _zero_rows(zbuf)
    for b in range(RQ // ZR):
        pltpu.sync_copy(zbuf, aggsh.at[pl.ds(s * RQ + b * ZR, ZR)])

    @pl.when(s == 0)
    def _():
        pltpu.sync_copy(zbuf.at[pl.ds(0, REM)], aggsh.at[pl.ds(NT * RQ, REM)])


def _copy_out(c, s, aggsh, out):
    pltpu.sync_copy(aggsh.at[pl.ds(s * RQ, RQ)], out.at[c, pl.ds(s * RQ, RQ)])

    @pl.when(s == 0)
    def _():
        pltpu.sync_copy(aggsh.at[pl.ds(NT * RQ, REM)], out.at[c, pl.ds(NT * RQ, REM)])


def _sc_conv_body(xa_h, xb_h, wbar_h, srcA_h, srcB_h, dstA_h, dstB_h, agg_o,
                  aggsh, srcvA, srcvB, dstvA, dstvB,
                  rowsA0, rowsB0, rowsA1, rowsB1, wch0, wch1, zbuf,
                  sem_gA0, sem_gB0, sem_gA1, sem_gB1, sem_w0, sem_w1,
                  sem_sA0, sem_sB0, sem_sA1, sem_sB1):
    c = lax.axis_index("c")
    s = lax.axis_index("s")
    _zero_agg(s, aggsh, zbuf)
    pltpu.sync_copy(srcA_h.at[s], srcvA)
    pltpu.sync_copy(srcB_h.at[s], srcvB)
    pltpu.sync_copy(dstA_h.at[s], dstvA)
    pltpu.sync_copy(dstB_h.at[s], dstvB)
    plsc.subcore_barrier()

    def issue(k, rowsA, rowsB, sem_gA, sem_gB, wch, sem_w):
        @pl.when(c == 0)
        def _():
            pltpu.async_copy(xa_h.at[srcvA.at[k]], rowsA, sem_gA)
            pltpu.async_copy(xa_h.at[srcvB.at[k]], rowsB, sem_gB)

        @pl.when(c == 1)
        def _():
            pltpu.async_copy(xb_h.at[srcvA.at[k]], rowsA, sem_gA)
            pltpu.async_copy(xb_h.at[srcvB.at[k]], rowsB, sem_gB)

        pltpu.async_copy(wbar_h.at[c, pl.ds(s * EPT + k * CH, CH)], wch, sem_w)

    def wait_in(k, rowsA, rowsB, sem_gA, sem_gB, wch, sem_w):
        @pl.when(c == 0)
        def _():
            pltpu.make_async_copy(xa_h.at[srcvA.at[k]], rowsA, sem_gA).wait()
            pltpu.make_async_copy(xa_h.at[srcvB.at[k]], rowsB, sem_gB).wait()

        @pl.when(c == 1)
        def _():
            pltpu.make_async_copy(xb_h.at[srcvA.at[k]], rowsA, sem_gA).wait()
            pltpu.make_async_copy(xb_h.at[srcvB.at[k]], rowsB, sem_gB).wait()

        pltpu.make_async_copy(
            wbar_h.at[c, pl.ds(s * EPT + k * CH, CH)], wch, sem_w).wait()

    def mul(rowsA, rowsB, wch):
        def mrow(i, _):
            for rr in range(4):
                r = i * 4 + rr
                for j in range(HF // 16):
                    sl = pl.ds(j * 16, 16)
                    rowsA[r, sl] = rowsA[r, sl] * wch[r, sl]
                    rowsB[r, sl] = rowsB[r, sl] * wch[r, pl.ds(HF + j * 16, 16)]
            return 0
        lax.fori_loop(0, CH // 4, mrow, 0)

    issue(0, rowsA0, rowsB0, sem_gA0, sem_gB0, wch0, sem_w0)

    def pair(i, _):
        k0 = 2 * i
        k1 = 2 * i + 1
        issue(k1, rowsA1, rowsB1, sem_gA1, sem_gB1, wch1, sem_w1)
        wait_in(k0, rowsA0, rowsB0, sem_gA0, sem_gB0, wch0, sem_w0)
        mul(rowsA0, rowsB0, wch0)
        csA0 = pltpu.async_copy(rowsA0, aggsh.at[dstvA.at[k0]], sem_sA0, add=True)
        csB0 = pltpu.async_copy(rowsB0, aggsh.at[dstvB.at[k0]], sem_sB0, add=True)
        wait_in(k1, rowsA1, rowsB1, sem_gA1, sem_gB1, wch1, sem_w1)
        mul(rowsA1, rowsB1, wch1)
        csA1 = pltpu.async_copy(rowsA1, aggsh.at[dstvA.at[k1]], sem_sA1, add=True)
        csB1 = pltpu.async_copy(rowsB1, aggsh.at[dstvB.at[k1]], sem_sB1, add=True)
        csA0.wait()
        csB0.wait()

        @pl.when(i < NCH // 2 - 1)
        def _():
            issue(k0 + 2, rowsA0, rowsB0, sem_gA0, sem_gB0, wch0, sem_w0)

        csA1.wait()
        csB1.wait()
        return 0

    lax.fori_loop(0, NCH // 2, pair, 0)
    plsc.subcore_barrier()
    _copy_out(c, s, aggsh, agg_o)


@functools.cache
def _build_sc_conv():
    return pl.kernel(
        _sc_conv_body,
        out_type=jax.ShapeDtypeStruct((2, N, HF), jnp.float32),
        mesh=_get_mesh(),
        compiler_params=pltpu.CompilerParams(use_tc_tiling_on_sc=False),
        scratch_types=[
            pltpu.VMEM_SHARED((N, HF), jnp.float32),
            pltpu.VMEM((NCH, CH), jnp.int32),
            pltpu.VMEM((NCH, CH), jnp.int32),
            pltpu.VMEM((NCH, CH), jnp.int32),
            pltpu.VMEM((NCH, CH), jnp.int32),
            pltpu.VMEM((CH, HF), jnp.float32),
            pltpu.VMEM((CH, HF), jnp.float32),
            pltpu.VMEM((CH, HF), jnp.float32),
            pltpu.VMEM((CH, HF), jnp.float32),
            pltpu.VMEM((CH, D), jnp.float32),
            pltpu.VMEM((CH, D), jnp.float32),
            pltpu.VMEM((ZR, HF), jnp.float32),
            pltpu.SemaphoreType.DMA,
            pltpu.SemaphoreType.DMA,
            pltpu.SemaphoreType.DMA,
            pltpu.SemaphoreType.DMA,
            pltpu.SemaphoreType.DMA,
            pltpu.SemaphoreType.DMA,
            pltpu.SemaphoreType.DMA,
            pltpu.SemaphoreType.DMA,
            pltpu.SemaphoreType.DMA,
            pltpu.SemaphoreType.DMA,
        ],
    )


def _sc_conv(xa, xb, wbar, srcA, srcB, dstA, dstB):
    return _build_sc_conv()(xa, xb, wbar, srcA, srcB, dstA, dstB)


def _sc_seg_body(wbar_h, srcA_h, srcB_h, s_o,
                 aggsh, srcvA, srcvB, wch0, wch1, bufA0, bufB0, bufA1, bufB1,
                 zbuf, sem_w0, sem_w1,
                 sem_sA0, sem_sB0, sem_sA1, sem_sB1):
    c = lax.axis_index("c")
    s = lax.axis_index("s")
    _zero_agg(s, aggsh, zbuf)
    pltpu.sync_copy(srcA_h.at[s], srcvA)
    pltpu.sync_copy(srcB_h.at[s], srcvB)
    plsc.subcore_barrier()

    def issue(k, wch, sem_w):
        pltpu.async_copy(wbar_h.at[c, pl.ds(s * EPT + k * CH, CH)], wch, sem_w)

    def wait_in(k, wch, sem_w):
        pltpu.make_async_copy(
            wbar_h.at[c, pl.ds(s * EPT + k * CH, CH)], wch, sem_w).wait()

    def split(wch, bufA, bufB):
        def srow(i, _):
            for rr in range(4):
                r = i * 4 + rr
                for j in range(HF // 16):
                    sl = pl.ds(j * 16, 16)
                    bufA[r, sl] = wch[r, sl]
                    bufB[r, sl] = wch[r, pl.ds(HF + j * 16, 16)]
            return 0
        lax.fori_loop(0, CH // 4, srow, 0)

    issue(0, wch0, sem_w0)

    def pair(i, _):
        k0 = 2 * i
        k1 = 2 * i + 1
        issue(k1, wch1, sem_w1)
        wait_in(k0, wch0, sem_w0)
        split(wch0, bufA0, bufB0)
        csA0 = pltpu.async_copy(bufA0, aggsh.at[srcvA.at[k0]], sem_sA0, add=True)
        csB0 = pltpu.async_copy(bufB0, aggsh.at[srcvB.at[k0]], sem_sB0, add=True)
        wait_in(k1, wch1, sem_w1)
        split(wch1, bufA1, bufB1)
        csA1 = pltpu.async_copy(bufA1, aggsh.at[srcvA.at[k1]], sem_sA1, add=True)
        csB1 = pltpu.async_copy(bufB1, aggsh.at[srcvB.at[k1]], sem_sB1, add=True)
        csA0.wait()
        csB0.wait()

        @pl.when(i < NCH // 2 - 1)
        def _():
            issue(k0 + 2, wch0, sem_w0)

        csA1.wait()
        csB1.wait()
        return 0

    lax.fori_loop(0, NCH // 2, pair, 0)
    plsc.subcore_barrier()
    _copy_out(c, s, aggsh, s_o)


@functools.cache
def _build_sc_seg():
    return pl.kernel(
        _sc_seg_body,
        out_type=jax.ShapeDtypeStruct((2, N, HF), jnp.float32),
        mesh=_get_mesh(),
        compiler_params=pltpu.CompilerParams(use_tc_tiling_on_sc=False),
        scratch_types=[
            pltpu.VMEM_SHARED((N, HF), jnp.float32),
            pltpu.VMEM((NCH, CH), jnp.int32),
            pltpu.VMEM((NCH, CH), jnp.int32),
            pltpu.VMEM((CH, D), jnp.float32),
            pltpu.VMEM((CH, D), jnp.float32),
            pltpu.VMEM((CH, HF), jnp.float32),
            pltpu.VMEM((CH, HF), jnp.float32),
            pltpu.VMEM((CH, HF), jnp.float32),
            pltpu.VMEM((CH, HF), jnp.float32),
            pltpu.VMEM((ZR, HF), jnp.float32),
            pltpu.SemaphoreType.DMA,
            pltpu.SemaphoreType.DMA,
            pltpu.SemaphoreType.DMA,
            pltpu.SemaphoreType.DMA,
            pltpu.SemaphoreType.DMA,
            pltpu.SemaphoreType.DMA,
        ],
    )


def _sc_seg(wbar, srcA, srcB):
    return _build_sc_seg()(wbar, srcA, srcB)


def kernel(x, pos, edge_index, edge_vec,
           W_sc_0, W_lin1_0, W_fc1_0, W_fc2_0, W_lin2_0,
           W_sc_1, W_lin1_1, W_fc1_1, W_fc2_1, W_lin2_1,
           W_sc_2, W_lin1_2, W_fc1_2, W_fc2_2, W_lin2_2):
    src = edge_index[0].astype(jnp.int32)
    dst = edge_index[1].astype(jnp.int32)
    srcA = src[:EH].reshape(NT, NCH, CH)
    srcB = src[EH:].reshape(NT, NCH, CH)
    dstA = dst[:EH].reshape(NT, NCH, CH)
    dstB = dst[EH:].reshape(NT, NCH, CH)
    evt = edge_vec.T
    wbar0 = _wbar_call(evt, W_fc1_0.T, W_fc2_0.T)
    wbar1 = _wbar_call(evt, W_fc1_1.T, W_fc2_1.T)
    wbar2 = _wbar_call(evt, W_fc1_2.T, W_fc2_2.T)
    s0, xa0, xb0 = _node_in_call(x, W_sc_0, W_lin1_0)
    agg0 = _sc_conv(xa0, xb0, wbar0, srcA, srcB, dstA, dstB)
    s1, xa1, xb1 = _epi_mid_call(s0, agg0, W_lin2_0, W_sc_1, W_lin1_1)
    agg1 = _sc_conv(xa1, xb1, wbar1, srcA, srcB, dstA, dstB)
    h1, xa2, xb2 = _epi_last_call(s1, agg1, W_lin2_1, W_lin1_2)
    s2 = _sc_seg(wbar2, srcA, srcB)
    return _final_call(h1, xa2, xb2, s2, W_sc_2, W_lin2_2)


# speedup vs baseline: 1.0017x; 1.0017x over previous
import functools

import numpy as np
import jax
import jax.numpy as jnp
from jax import lax
from jax.experimental import pallas as pl
from jax.experimental.pallas import tpu as pltpu
from jax.experimental.pallas import tpu_sc as plsc

N = 10000
E = 320000
D = 128
NB = 10
RN = 64
MAX_R = 3.5
HF = 64
NT = 16
EH = E // 2
EPT = EH // NT
CH = 40
NCH = EPT // CH
RQ = 624
REM = N - NT * RQ
ZR = 156
SIN = float(np.sin(np.pi / 8.0))
COS = float(np.cos(np.pi / 8.0))
INV_SQRT_D = float(1.0 / np.sqrt(D))
INV_SQRT_NB = float(1.0 / np.sqrt(NB))
INV_SQRT_RN = float(1.0 / np.sqrt(RN))
INV_SQRT_NEI = float(1.0 / np.sqrt(32.0))
INV_SQRT_N = float(1.0 / np.sqrt(10000.0))

@functools.cache
def _get_mesh():
    return plsc.VectorSubcoreMesh(core_axis_name="c", subcore_axis_name="s")


_BP = 3200


def _wbar_body(evtA, evtB, f1, f2, o):
    v = jnp.concatenate([evtA[...], evtB[...]], axis=1)
    el2 = jnp.sum(v * v, axis=0, keepdims=True)
    elen = jnp.sqrt(el2)
    step = MAX_R / (NB - 1)
    centers = lax.broadcasted_iota(jnp.int32, (NB, 1), 0).astype(jnp.float32) * step
    diff = (elen - centers) / step
    eln = jnp.exp(-diff * diff) * (float(np.sqrt(NB)) / 1.12)
    u = 2.0 * (elen / MAX_R - 1.0)
    cut = (1.0 - jnp.cos(jnp.pi * u)) * 0.5
    cut = jnp.where(u > 0.0, 0.0, cut)
    cut = jnp.where(u < -1.0, 1.0, cut)
    elnb = eln.astype(jnp.bfloat16)
    ht = jax.nn.silu(
        jnp.dot(f1[...].astype(jnp.bfloat16), elnb,
                preferred_element_type=jnp.float32) * INV_SQRT_NB)
    ht = ht * (cut * INV_SQRT_RN)
    wt = jnp.dot(f2[...].astype(jnp.bfloat16), ht.astype(jnp.bfloat16),
                 preferred_element_type=jnp.float32)
    wb = wt.T
    pA = wb[:_BP]
    pB = wb[_BP:]
    o[0] = jnp.concatenate([pA[:, :HF], pB[:, :HF]], axis=1)
    o[1] = jnp.concatenate([pA[:, HF:], pB[:, HF:]], axis=1)


def _wbar_call(evt, f1t, f2t):
    nblk = EH // _BP
    return pl.pallas_call(
        _wbar_body,
        grid=(nblk,),
        in_specs=[
            pl.BlockSpec((3, _BP), lambda i: (0, i)),
            pl.BlockSpec((3, _BP), lambda i, n=nblk: (0, i + n)),
            pl.BlockSpec((RN, NB), lambda i: (0, 0)),
            pl.BlockSpec((D, RN), lambda i: (0, 0)),
        ],
        out_specs=pl.BlockSpec((2, _BP, D), lambda i: (0, i, 0)),
        out_shape=jax.ShapeDtypeStruct((2, EH, D), jnp.float32),
    )(evt, evt, f1t, f2t)


_BN = 2000


def _node_in_body(x_r, wsc_r, wl1_r, s_o, xa_o, xb_o):
    xv = x_r[...]
    s_o[...] = jnp.dot(xv, wsc_r[...], preferred_element_type=jnp.float32) * INV_SQRT_D
    xl = jnp.dot(xv, wl1_r[...], preferred_element_type=jnp.float32) * INV_SQRT_D
    xa_o[...] = xl[:, :HF]
    xb_o[...] = xl[:, HF:]


def _node_in_call(x, wsc, wl1):
    wspec = pl.BlockSpec((D, D), lambda i: (0, 0))
    return pl.pallas_call(
        _node_in_body,
        grid=(N // _BN,),
        in_specs=[pl.BlockSpec((_BN, D), lambda i: (i, 0)), wspec, wspec],
        out_specs=[pl.BlockSpec((_BN, D), lambda i: (i, 0)),
                   pl.BlockSpec((_BN, HF), lambda i: (i, 0)),
                   pl.BlockSpec((_BN, HF), lambda i: (i, 0))],
        out_shape=[jax.ShapeDtypeStruct((N, D), jnp.float32),
                   jax.ShapeDtypeStruct((N, HF), jnp.float32),
                   jax.ShapeDtypeStruct((N, HF), jnp.float32)],
    )(x, wsc, wl1)


def _epi_mid_body(s_r, agg_r, wl2_r, wsc_r, wl1_r, s_o, xa_o, xb_o):
    a = jnp.concatenate([agg_r[0], agg_r[1]], axis=1)
    out = (jnp.dot(a, wl2_r[...], preferred_element_type=jnp.float32)
           * (INV_SQRT_NEI * INV_SQRT_D))
    h = jax.nn.silu(SIN * s_r[...] + COS * out)
    s_o[...] = jnp.dot(h, wsc_r[...], preferred_element_type=jnp.float32) * INV_SQRT_D
    xl = jnp.dot(h, wl1_r[...], preferred_element_type=jnp.float32) * INV_SQRT_D
    xa_o[...] = xl[:, :HF]
    xb_o[...] = xl[:, HF:]


def _epi_mid_call(s, agg, wl2, wsc, wl1):
    wspec = pl.BlockSpec((D, D), lambda i: (0, 0))
    return pl.pallas_call(
        _epi_mid_body,
        grid=(N // _BN,),
        in_specs=[pl.BlockSpec((_BN, D), lambda i: (i, 0)),
                  pl.BlockSpec((2, _BN, HF), lambda i: (0, i, 0)),
                  wspec, wspec, wspec],
        out_specs=[pl.BlockSpec((_BN, D), lambda i: (i, 0)),
                   pl.BlockSpec((_BN, HF), lambda i: (i, 0)),
                   pl.BlockSpec((_BN, HF), lambda i: (i, 0))],
        out_shape=[jax.ShapeDtypeStruct((N, D), jnp.float32),
                   jax.ShapeDtypeStruct((N, HF), jnp.float32),
                   jax.ShapeDtypeStruct((N, HF), jnp.float32)],
    )(s, agg, wl2, wsc, wl1)


def _epi_last_body(s_r, agg_r, wl2_r, wl1_r, h_o, xa_o, xb_o):
    a = jnp.concatenate([agg_r[0], agg_r[1]], axis=1)
    out = (jnp.dot(a, wl2_r[...], preferred_element_type=jnp.float32)
           * (INV_SQRT_NEI * INV_SQRT_D))
    h = jax.nn.silu(SIN * s_r[...] + COS * out)
    h_o[...] = h
    xl = jnp.dot(h, wl1_r[...], preferred_element_type=jnp.float32) * INV_SQRT_D
    xa_o[...] = xl[:, :HF]
    xb_o[...] = xl[:, HF:]


def _epi_last_call(s, agg, wl2, wl1):
    wspec = pl.BlockSpec((D, D), lambda i: (0, 0))
    return pl.pallas_call(
        _epi_last_body,
        grid=(N // _BN,),
        in_specs=[pl.BlockSpec((_BN, D), lambda i: (i, 0)),
                  pl.BlockSpec((2, _BN, HF), lambda i: (0, i, 0)),
                  wspec, wspec],
        out_specs=[pl.BlockSpec((_BN, D), lambda i: (i, 0)),
                   pl.BlockSpec((_BN, HF), lambda i: (i, 0)),
                   pl.BlockSpec((_BN, HF), lambda i: (i, 0))],
        out_shape=[jax.ShapeDtypeStruct((N, D), jnp.float32),
                   jax.ShapeDtypeStruct((N, HF), jnp.float32),
                   jax.ShapeDtypeStruct((N, HF), jnp.float32)],
    )(s, agg, wl2, wl1)


def _final_body(h_r, xa_r, xb_r, s2_r, wsc_r, wl2_r, o):
    hsum = jnp.sum(h_r[...], axis=0, keepdims=True)
    x0 = jnp.sum(xa_r[...] * s2_r[0], axis=0, keepdims=True)
    x1 = jnp.sum(xb_r[...] * s2_r[1], axis=0, keepdims=True)
    xsum = jnp.concatenate([x0, x1], axis=1)
    ssum = jnp.dot(hsum, wsc_r[...], preferred_element_type=jnp.float32) * INV_SQRT_D
    osum = (jnp.dot(xsum, wl2_r[...], preferred_element_type=jnp.float32)
            * (INV_SQRT_NEI * INV_SQRT_D))
    o[...] = (SIN * ssum + COS * osum) * INV_SQRT_N


def _final_call(h1, xa2, xb2, s2, wsc2, wl22):
    return pl.pallas_call(
        _final_body,
        grid=(1,),
        in_specs=[pl.BlockSpec((N, D), lambda i: (0, 0)),
                  pl.BlockSpec((N, HF), lambda i: (0, 0)),
                  pl.BlockSpec((N, HF), lambda i: (0, 0)),
                  pl.BlockSpec((2, N, HF), lambda i: (0, 0, 0)),
                  pl.BlockSpec((D, 1), lambda i: (0, 0)),
                  pl.BlockSpec((D, 1), lambda i: (0, 0))],
        out_specs=pl.BlockSpec((1, 1), lambda i: (0, 0)),
        out_shape=jax.ShapeDtypeStruct((1, 1), jnp.float32),
    )(h1, xa2, xb2, s2, wsc2, wl22)


def _zero_rows(zbuf):
    def zrow(i, _):
        for j in range(HF // 16):
            zbuf[i, pl.ds(j * 16, 16)] = jnp.zeros((16,), jnp.float32)
        return 0
    lax.fori_loop(0, ZR, zrow, 0)


def _zero_agg(s, aggsh, zbuf):
    _zero_rows(zbuf)
    for b in range(RQ // ZR):
        pltpu.sync_copy(zbuf, aggsh.at[pl.ds(s * RQ + b * ZR, ZR)])

    @pl.when(s == 0)
    def _():
        pltpu.sync_copy(zbuf.at[pl.ds(0, REM)], aggsh.at[pl.ds(NT * RQ, REM)])


def _copy_out(c, s, aggsh, out):
    pltpu.sync_copy(aggsh.at[pl.ds(s * RQ, RQ)], out.at[c, pl.ds(s * RQ, RQ)])

    @pl.when(s == 0)
    def _():
        pltpu.sync_copy(aggsh.at[pl.ds(NT * RQ, REM)], out.at[c, pl.ds(NT * RQ, REM)])


CE = 2 * CH


def _sc_conv_body(xa_h, xb_h, wbar_h, src_h, dst_h, agg_o,
                  aggsh, srcv, dstv, rows0, rows1, wch0, wch1, zbuf,
                  sem_g0, sem_g1, sem_w0, sem_w1, sem_s0, sem_s1):
    c = lax.axis_index("c")
    s = lax.axis_index("s")
    _zero_agg(s, aggsh, zbuf)
    pltpu.sync_copy(src_h.at[s], srcv)
    pltpu.sync_copy(dst_h.at[s], dstv)
    plsc.subcore_barrier()

    def issue(k, rows, sem_g, wch, sem_w):
        @pl.when(c == 0)
        def _():
            pltpu.async_copy(xa_h.at[srcv.at[k]], rows, sem_g)

        @pl.when(c == 1)
        def _():
            pltpu.async_copy(xb_h.at[srcv.at[k]], rows, sem_g)

        pltpu.async_copy(wbar_h.at[c, pl.ds(s * EPT + k * CH, CH)], wch, sem_w)

    def wait_in(k, rows, sem_g, wch, sem_w):
        @pl.when(c == 0)
        def _():
            pltpu.make_async_copy(xa_h.at[srcv.at[k]], rows, sem_g).wait()

        @pl.when(c == 1)
        def _():
            pltpu.make_async_copy(xb_h.at[srcv.at[k]], rows, sem_g).wait()

        pltpu.make_async_copy(
            wbar_h.at[c, pl.ds(s * EPT + k * CH, CH)], wch, sem_w).wait()

    def mul(rows, wch):
        def mrow(i, _):
            for rr in range(4):
                q = i * 4 + rr
                for j in range(HF // 16):
                    sl = pl.ds(j * 16, 16)
                    rows[q, sl] = rows[q, sl] * wch[q, sl]
                    rows[CH + q, sl] = rows[CH + q, sl] * wch[q, pl.ds(HF + j * 16, 16)]
            return 0
        lax.fori_loop(0, CH // 4, mrow, 0)

    issue(0, rows0, sem_g0, wch0, sem_w0)

    def pair(i, _):
        k0 = 2 * i
        k1 = 2 * i + 1
        issue(k1, rows1, sem_g1, wch1, sem_w1)
        wait_in(k0, rows0, sem_g0, wch0, sem_w0)
        mul(rows0, wch0)
        cs0 = pltpu.async_copy(rows0, aggsh.at[dstv.at[k0]], sem_s0, add=True)
        wait_in(k1, rows1, sem_g1, wch1, sem_w1)
        mul(rows1, wch1)
        cs1 = pltpu.async_copy(rows1, aggsh.at[dstv.at[k1]], sem_s1, add=True)
        cs0.wait()

        @pl.when(i < NCH // 2 - 1)
        def _():
            issue(k0 + 2, rows0, sem_g0, wch0, sem_w0)

        cs1.wait()
        return 0

    lax.fori_loop(0, NCH // 2, pair, 0)
    plsc.subcore_barrier()
    _copy_out(c, s, aggsh, agg_o)


@functools.cache
def _build_sc_conv():
    return pl.kernel(
        _sc_conv_body,
        out_type=jax.ShapeDtypeStruct((2, N, HF), jnp.float32),
        mesh=_get_mesh(),
        compiler_params=pltpu.CompilerParams(use_tc_tiling_on_sc=False),
        scratch_types=[
            pltpu.VMEM_SHARED((N, HF), jnp.float32),
            pltpu.VMEM((NCH, CE), jnp.int32),
            pltpu.VMEM((NCH, CE), jnp.int32),
            pltpu.VMEM((CE, HF), jnp.float32),
            pltpu.VMEM((CE, HF), jnp.float32),
            pltpu.VMEM((CH, D), jnp.float32),
            pltpu.VMEM((CH, D), jnp.float32),
            pltpu.VMEM((ZR, HF), jnp.float32),
            pltpu.SemaphoreType.DMA,
            pltpu.SemaphoreType.DMA,
            pltpu.SemaphoreType.DMA,
            pltpu.SemaphoreType.DMA,
            pltpu.SemaphoreType.DMA,
            pltpu.SemaphoreType.DMA,
        ],
    )


def _sc_conv(xa, xb, wbar, src3, dst3):
    return _build_sc_conv()(xa, xb, wbar, src3, dst3)


def _sc_seg_body(wbar_h, src_h, s_o,
                 aggsh, srcv, wch0, wch1, buf0, buf1,
                 zbuf, sem_w0, sem_w1, sem_s0, sem_s1):
    c = lax.axis_index("c")
    s = lax.axis_index("s")
    _zero_agg(s, aggsh, zbuf)
    pltpu.sync_copy(src_h.at[s], srcv)
    plsc.subcore_barrier()

    def issue(k, wch, sem_w):
        pltpu.async_copy(wbar_h.at[c, pl.ds(s * EPT + k * CH, CH)], wch, sem_w)

    def wait_in(k, wch, sem_w):
        pltpu.make_async_copy(
            wbar_h.at[c, pl.ds(s * EPT + k * CH, CH)], wch, sem_w).wait()

    def split(wch, buf):
        def srow(i, _):
            for rr in range(4):
                q = i * 4 + rr
                for j in range(HF // 16):
                    sl = pl.ds(j * 16, 16)
                    buf[q, sl] = wch[q, sl]
                    buf[CH + q, sl] = wch[q, pl.ds(HF + j * 16, 16)]
            return 0
        lax.fori_loop(0, CH // 4, srow, 0)

    issue(0, wch0, sem_w0)

    def pair(i, _):
        k0 = 2 * i
        k1 = 2 * i + 1
        issue(k1, wch1, sem_w1)
        wait_in(k0, wch0, sem_w0)
        split(wch0, buf0)
        cs0 = pltpu.async_copy(buf0, aggsh.at[srcv.at[k0]], sem_s0, add=True)
        wait_in(k1, wch1, sem_w1)
        split(wch1, buf1)
        cs1 = pltpu.async_copy(buf1, aggsh.at[srcv.at[k1]], sem_s1, add=True)
        cs0.wait()

        @pl.when(i < NCH // 2 - 1)
        def _():
            issue(k0 + 2, wch0, sem_w0)

        cs1.wait()
        return 0

    lax.fori_loop(0, NCH // 2, pair, 0)
    plsc.subcore_barrier()
    _copy_out(c, s, aggsh, s_o)


@functools.cache
def _build_sc_seg():
    return pl.kernel(
        _sc_seg_body,
        out_type=jax.ShapeDtypeStruct((2, N, HF), jnp.float32),
        mesh=_get_mesh(),
        compiler_params=pltpu.CompilerParams(use_tc_tiling_on_sc=False),
        scratch_types=[
            pltpu.VMEM_SHARED((N, HF), jnp.float32),
            pltpu.VMEM((NCH, CE), jnp.int32),
            pltpu.VMEM((CH, D), jnp.float32),
            pltpu.VMEM((CH, D), jnp.float32),
            pltpu.VMEM((CE, HF), jnp.float32),
            pltpu.VMEM((CE, HF), jnp.float32),
            pltpu.VMEM((ZR, HF), jnp.float32),
            pltpu.SemaphoreType.DMA,
            pltpu.SemaphoreType.DMA,
            pltpu.SemaphoreType.DMA,
            pltpu.SemaphoreType.DMA,
        ],
    )


def _sc_seg(wbar, src3):
    return _build_sc_seg()(wbar, src3)


def kernel(x, pos, edge_index, edge_vec,
           W_sc_0, W_lin1_0, W_fc1_0, W_fc2_0, W_lin2_0,
           W_sc_1, W_lin1_1, W_fc1_1, W_fc2_1, W_lin2_1,
           W_sc_2, W_lin1_2, W_fc1_2, W_fc2_2, W_lin2_2):
    src = edge_index[0].astype(jnp.int32)
    dst = edge_index[1].astype(jnp.int32)
    src3 = jnp.concatenate(
        [src[:EH].reshape(NT, NCH, CH), src[EH:].reshape(NT, NCH, CH)], axis=2)
    dst3 = jnp.concatenate(
        [dst[:EH].reshape(NT, NCH, CH), dst[EH:].reshape(NT, NCH, CH)], axis=2)
    evt = edge_vec.T
    wbar0 = _wbar_call(evt, W_fc1_0.T, W_fc2_0.T)
    wbar1 = _wbar_call(evt, W_fc1_1.T, W_fc2_1.T)
    wbar2 = _wbar_call(evt, W_fc1_2.T, W_fc2_2.T)
    s0, xa0, xb0 = _node_in_call(x, W_sc_0, W_lin1_0)
    agg0 = _sc_conv(xa0, xb0, wbar0, src3, dst3)
    s1, xa1, xb1 = _epi_mid_call(s0, agg0, W_lin2_0, W_sc_1, W_lin1_1)
    agg1 = _sc_conv(xa1, xb1, wbar1, src3, dst3)
    h1, xa2, xb2 = _epi_last_call(s1, agg1, W_lin2_1, W_lin1_2)
    s2 = _sc_seg(wbar2, src3)
    return _final_call(h1, xa2, xb2, s2, W_sc_2, W_lin2_2)


# speedup vs baseline: 1.0437x; 1.0420x over previous
import functools

import numpy as np
import jax
import jax.numpy as jnp
from jax import lax
from jax.experimental import pallas as pl
from jax.experimental.pallas import tpu as pltpu
from jax.experimental.pallas import tpu_sc as plsc

N = 10000
E = 320000
D = 128
NB = 10
RN = 64
MAX_R = 3.5
HF = 64
NT = 16
EPT = E // NT
CH = 80
NCH = EPT // CH
RQ = 624
REM = N - NT * RQ
ZR = 156
SIN = float(np.sin(np.pi / 8.0))
COS = float(np.cos(np.pi / 8.0))
INV_SQRT_D = float(1.0 / np.sqrt(D))
INV_SQRT_NB = float(1.0 / np.sqrt(NB))
INV_SQRT_RN = float(1.0 / np.sqrt(RN))
INV_SQRT_NEI = float(1.0 / np.sqrt(32.0))
INV_SQRT_N = float(1.0 / np.sqrt(10000.0))

@functools.cache
def _get_mesh():
    return plsc.VectorSubcoreMesh(core_axis_name="c", subcore_axis_name="s")


_BE = 6400


def _wbar_body(evt, f1, f2, o):
    v = evt[...]
    el2 = jnp.sum(v * v, axis=0, keepdims=True)
    elen = jnp.sqrt(el2)
    step = MAX_R / (NB - 1)
    centers = lax.broadcasted_iota(jnp.int32, (NB, 1), 0).astype(jnp.float32) * step
    diff = (elen - centers) / step
    eln = jnp.exp(-diff * diff) * (float(np.sqrt(NB)) / 1.12)
    u = 2.0 * (elen / MAX_R - 1.0)
    cut = (1.0 - jnp.cos(jnp.pi * u)) * 0.5
    cut = jnp.where(u > 0.0, 0.0, cut)
    cut = jnp.where(u < -1.0, 1.0, cut)
    elnb = eln.astype(jnp.bfloat16)
    ht = jax.nn.silu(
        jnp.dot(f1[...].astype(jnp.bfloat16), elnb,
                preferred_element_type=jnp.float32) * INV_SQRT_NB)
    ht = ht * (cut * INV_SQRT_RN)
    wt = jnp.dot(f2[...].astype(jnp.bfloat16), ht.astype(jnp.bfloat16),
                 preferred_element_type=jnp.float32)
    wb = wt.T
    o[0] = wb[:, :HF]
    o[1] = wb[:, HF:]


def _wbar_call(evt, f1t, f2t):
    return pl.pallas_call(
        _wbar_body,
        grid=(E // _BE,),
        in_specs=[
            pl.BlockSpec((3, _BE), lambda i: (0, i)),
            pl.BlockSpec((RN, NB), lambda i: (0, 0)),
            pl.BlockSpec((D, RN), lambda i: (0, 0)),
        ],
        out_specs=pl.BlockSpec((2, _BE, HF), lambda i: (0, i, 0)),
        out_shape=jax.ShapeDtypeStruct((2, E, HF), jnp.float32),
    )(evt, f1t, f2t)


_BN = 2000


def _node_in_body(x_r, wsc_r, wl1_r, s_o, xa_o, xb_o):
    xv = x_r[...]
    s_o[...] = jnp.dot(xv, wsc_r[...], preferred_element_type=jnp.float32) * INV_SQRT_D
    xl = jnp.dot(xv, wl1_r[...], preferred_element_type=jnp.float32) * INV_SQRT_D
    xa_o[...] = xl[:, :HF]
    xb_o[...] = xl[:, HF:]


def _node_in_call(x, wsc, wl1):
    wspec = pl.BlockSpec((D, D), lambda i: (0, 0))
    return pl.pallas_call(
        _node_in_body,
        grid=(N // _BN,),
        in_specs=[pl.BlockSpec((_BN, D), lambda i: (i, 0)), wspec, wspec],
        out_specs=[pl.BlockSpec((_BN, D), lambda i: (i, 0)),
                   pl.BlockSpec((_BN, HF), lambda i: (i, 0)),
                   pl.BlockSpec((_BN, HF), lambda i: (i, 0))],
        out_shape=[jax.ShapeDtypeStruct((N, D), jnp.float32),
                   jax.ShapeDtypeStruct((N, HF), jnp.float32),
                   jax.ShapeDtypeStruct((N, HF), jnp.float32)],
    )(x, wsc, wl1)


def _epi_mid_body(s_r, agg_r, wl2_r, wsc_r, wl1_r, s_o, xa_o, xb_o):
    a = jnp.concatenate([agg_r[0], agg_r[1]], axis=1)
    out = (jnp.dot(a, wl2_r[...], preferred_element_type=jnp.float32)
           * (INV_SQRT_NEI * INV_SQRT_D))
    h = jax.nn.silu(SIN * s_r[...] + COS * out)
    s_o[...] = jnp.dot(h, wsc_r[...], preferred_element_type=jnp.float32) * INV_SQRT_D
    xl = jnp.dot(h, wl1_r[...], preferred_element_type=jnp.float32) * INV_SQRT_D
    xa_o[...] = xl[:, :HF]
    xb_o[...] = xl[:, HF:]


def _epi_mid_call(s, agg, wl2, wsc, wl1):
    wspec = pl.BlockSpec((D, D), lambda i: (0, 0))
    return pl.pallas_call(
        _epi_mid_body,
        grid=(N // _BN,),
        in_specs=[pl.BlockSpec((_BN, D), lambda i: (i, 0)),
                  pl.BlockSpec((2, _BN, HF), lambda i: (0, i, 0)),
                  wspec, wspec, wspec],
        out_specs=[pl.BlockSpec((_BN, D), lambda i: (i, 0)),
                   pl.BlockSpec((_BN, HF), lambda i: (i, 0)),
                   pl.BlockSpec((_BN, HF), lambda i: (i, 0))],
        out_shape=[jax.ShapeDtypeStruct((N, D), jnp.float32),
                   jax.ShapeDtypeStruct((N, HF), jnp.float32),
                   jax.ShapeDtypeStruct((N, HF), jnp.float32)],
    )(s, agg, wl2, wsc, wl1)


def _epi_last_body(s_r, agg_r, wl2_r, wl1_r, h_o, xa_o, xb_o):
    a = jnp.concatenate([agg_r[0], agg_r[1]], axis=1)
    out = (jnp.dot(a, wl2_r[...], preferred_element_type=jnp.float32)
           * (INV_SQRT_NEI * INV_SQRT_D))
    h = jax.nn.silu(SIN * s_r[...] + COS * out)
    h_o[...] = h
    xl = jnp.dot(h, wl1_r[...], preferred_element_type=jnp.float32) * INV_SQRT_D
    xa_o[...] = xl[:, :HF]
    xb_o[...] = xl[:, HF:]


def _epi_last_call(s, agg, wl2, wl1):
    wspec = pl.BlockSpec((D, D), lambda i: (0, 0))
    return pl.pallas_call(
        _epi_last_body,
        grid=(N // _BN,),
        in_specs=[pl.BlockSpec((_BN, D), lambda i: (i, 0)),
                  pl.BlockSpec((2, _BN, HF), lambda i: (0, i, 0)),
                  wspec, wspec],
        out_specs=[pl.BlockSpec((_BN, D), lambda i: (i, 0)),
                   pl.BlockSpec((_BN, HF), lambda i: (i, 0)),
                   pl.BlockSpec((_BN, HF), lambda i: (i, 0))],
        out_shape=[jax.ShapeDtypeStruct((N, D), jnp.float32),
                   jax.ShapeDtypeStruct((N, HF), jnp.float32),
                   jax.ShapeDtypeStruct((N, HF), jnp.float32)],
    )(s, agg, wl2, wl1)


def _final_body(h_r, xa_r, xb_r, s2_r, wsc_r, wl2_r, o):
    hsum = jnp.sum(h_r[...], axis=0, keepdims=True)
    x0 = jnp.sum(xa_r[...] * s2_r[0], axis=0, keepdims=True)
    x1 = jnp.sum(xb_r[...] * s2_r[1], axis=0, keepdims=True)
    xsum = jnp.concatenate([x0, x1], axis=1)
    ssum = jnp.dot(hsum, wsc_r[...], preferred_element_type=jnp.float32) * INV_SQRT_D
    osum = (jnp.dot(xsum, wl2_r[...], preferred_element_type=jnp.float32)
            * (INV_SQRT_NEI * INV_SQRT_D))
    o[...] = (SIN * ssum + COS * osum) * INV_SQRT_N


def _final_call(h1, xa2, xb2, s2, wsc2, wl22):
    return pl.pallas_call(
        _final_body,
        grid=(1,),
        in_specs=[pl.BlockSpec((N, D), lambda i: (0, 0)),
                  pl.BlockSpec((N, HF), lambda i: (0, 0)),
                  pl.BlockSpec((N, HF), lambda i: (0, 0)),
                  pl.BlockSpec((2, N, HF), lambda i: (0, 0, 0)),
                  pl.BlockSpec((D, 1), lambda i: (0, 0)),
                  pl.BlockSpec((D, 1), lambda i: (0, 0))],
        out_specs=pl.BlockSpec((1, 1), lambda i: (0, 0)),
        out_shape=jax.ShapeDtypeStruct((1, 1), jnp.float32),
    )(h1, xa2, xb2, s2, wsc2, wl22)


def _zero_rows(zbuf):
    def zrow(i, _):
        for j in range(HF // 16):
            zbuf[i, pl.ds(j * 16, 16)] = jnp.zeros((16,), jnp.float32)
        return 0
    lax.fori_loop(0, ZR, zrow, 0)


def _sc_conv_body(xa_h, xb_h, wbar_h, src_h, dst_h, agg_o,
                  aggsh, srcv, dstv, rows0, rows1, wch0, wch1, zbuf,
                  sem_g0, sem_g1, sem_w0, sem_w1, sem_s0, sem_s1):
    c = lax.axis_index("c")
    s = lax.axis_index("s")
    _zero_rows(zbuf)
    for b in range(RQ // ZR):
        pltpu.sync_copy(zbuf, aggsh.at[pl.ds(s * RQ + b * ZR, ZR)])

    @pl.when(s == 0)
    def _():
        pltpu.sync_copy(zbuf.at[pl.ds(0, REM)], aggsh.at[pl.ds(NT * RQ, REM)])

    pltpu.sync_copy(src_h.at[s], srcv)
    pltpu.sync_copy(dst_h.at[s], dstv)
    plsc.subcore_barrier()

    def issue(k, rows, sem_g, wch, sem_w):
        @pl.when(c == 0)
        def _():
            pltpu.async_copy(xa_h.at[srcv.at[k]], rows, sem_g)

        @pl.when(c == 1)
        def _():
            pltpu.async_copy(xb_h.at[srcv.at[k]], rows, sem_g)

        pltpu.async_copy(wbar_h.at[c, pl.ds(s * EPT + k * CH, CH)], wch, sem_w)

    def wait_in(k, rows, sem_g, wch, sem_w):
        @pl.when(c == 0)
        def _():
            pltpu.make_async_copy(xa_h.at[srcv.at[k]], rows, sem_g).wait()

        @pl.when(c == 1)
        def _():
            pltpu.make_async_copy(xb_h.at[srcv.at[k]], rows, sem_g).wait()

        pltpu.make_async_copy(
            wbar_h.at[c, pl.ds(s * EPT + k * CH, CH)], wch, sem_w).wait()

    def mul(rows, wch):
        def mrow(i, _):
            for rr in range(4):
                r = i * 4 + rr
                for j in range(HF // 16):
                    sl = pl.ds(j * 16, 16)
                    rows[r, sl] = rows[r, sl] * wch[r, sl]
            return 0
        lax.fori_loop(0, CH // 4, mrow, 0)

    issue(0, rows0, sem_g0, wch0, sem_w0)

    def pair(i, _):
        k0 = 2 * i
        k1 = 2 * i + 1
        issue(k1, rows1, sem_g1, wch1, sem_w1)
        wait_in(k0, rows0, sem_g0, wch0, sem_w0)
        mul(rows0, wch0)
        cs0 = pltpu.async_copy(rows0, aggsh.at[dstv.at[k0]], sem_s0, add=True)
        wait_in(k1, rows1, sem_g1, wch1, sem_w1)
        mul(rows1, wch1)
        cs1 = pltpu.async_copy(rows1, aggsh.at[dstv.at[k1]], sem_s1, add=True)
        cs0.wait()

        @pl.when(i < NCH // 2 - 1)
        def _():
            issue(k0 + 2, rows0, sem_g0, wch0, sem_w0)

        cs1.wait()
        return 0

    lax.fori_loop(0, NCH // 2, pair, 0)
    plsc.subcore_barrier()
    pltpu.sync_copy(aggsh.at[pl.ds(s * RQ, RQ)], agg_o.at[c, pl.ds(s * RQ, RQ)])

    @pl.when(s == 0)
    def _():
        pltpu.sync_copy(aggsh.at[pl.ds(NT * RQ, REM)], agg_o.at[c, pl.ds(NT * RQ, REM)])


@functools.cache
def _build_sc_conv():
    return pl.kernel(
        _sc_conv_body,
        out_type=jax.ShapeDtypeStruct((2, N, HF), jnp.float32),
        mesh=_get_mesh(),
        compiler_params=pltpu.CompilerParams(use_tc_tiling_on_sc=False),
        scratch_types=[
            pltpu.VMEM_SHARED((N, HF), jnp.float32),
            pltpu.VMEM((NCH, CH), jnp.int32),
            pltpu.VMEM((NCH, CH), jnp.int32),
            pltpu.VMEM((CH, HF), jnp.float32),
            pltpu.VMEM((CH, HF), jnp.float32),
            pltpu.VMEM((CH, HF), jnp.float32),
            pltpu.VMEM((CH, HF), jnp.float32),
            pltpu.VMEM((ZR, HF), jnp.float32),
            pltpu.SemaphoreType.DMA,
            pltpu.SemaphoreType.DMA,
            pltpu.SemaphoreType.DMA,
            pltpu.SemaphoreType.DMA,
            pltpu.SemaphoreType.DMA,
            pltpu.SemaphoreType.DMA,
        ],
    )


def _sc_conv(xa, xb, wbar, src3, dst3):
    return _build_sc_conv()(xa, xb, wbar, src3, dst3)


def _sc_seg_body(wbar_h, src_h, s_o, aggsh, srcv, wch0, wch1, zbuf,
                 sem_w0, sem_w1, sem_s0, sem_s1):
    c = lax.axis_index("c")
    s = lax.axis_index("s")
    _zero_rows(zbuf)
    for b in range(RQ // ZR):
        pltpu.sync_copy(zbuf, aggsh.at[pl.ds(s * RQ + b * ZR, ZR)])

    @pl.when(s == 0)
    def _():
        pltpu.sync_copy(zbuf.at[pl.ds(0, REM)], aggsh.at[pl.ds(NT * RQ, REM)])

    pltpu.sync_copy(src_h.at[s], srcv)
    plsc.subcore_barrier()

    def issue(k, wch, sem_w):
        pltpu.async_copy(wbar_h.at[c, pl.ds(s * EPT + k * CH, CH)], wch, sem_w)

    def wait_in(k, wch, sem_w):
        pltpu.make_async_copy(
            wbar_h.at[c, pl.ds(s * EPT + k * CH, CH)], wch, sem_w).wait()

    issue(0, wch0, sem_w0)

    def pair(i, _):
        k0 = 2 * i
        k1 = 2 * i + 1
        issue(k1, wch1, sem_w1)
        wait_in(k0, wch0, sem_w0)
        cs0 = pltpu.async_copy(wch0, aggsh.at[srcv.at[k0]], sem_s0, add=True)
        wait_in(k1, wch1, sem_w1)
        cs1 = pltpu.async_copy(wch1, aggsh.at[srcv.at[k1]], sem_s1, add=True)
        cs0.wait()

        @pl.when(i < NCH // 2 - 1)
        def _():
            issue(k0 + 2, wch0, sem_w0)

        cs1.wait()
        return 0

    lax.fori_loop(0, NCH // 2, pair, 0)
    plsc.subcore_barrier()
    pltpu.sync_copy(aggsh.at[pl.ds(s * RQ, RQ)], s_o.at[c, pl.ds(s * RQ, RQ)])

    @pl.when(s == 0)
    def _():
        pltpu.sync_copy(aggsh.at[pl.ds(NT * RQ, REM)], s_o.at[c, pl.ds(NT * RQ, REM)])


@functools.cache
def _build_sc_seg():
    return pl.kernel(
        _sc_seg_body,
        out_type=jax.ShapeDtypeStruct((2, N, HF), jnp.float32),
        mesh=_get_mesh(),
        compiler_params=pltpu.CompilerParams(use_tc_tiling_on_sc=False),
        scratch_types=[
            pltpu.VMEM_SHARED((N, HF), jnp.float32),
            pltpu.VMEM((NCH, CH), jnp.int32),
            pltpu.VMEM((CH, HF), jnp.float32),
            pltpu.VMEM((CH, HF), jnp.float32),
            pltpu.VMEM((ZR, HF), jnp.float32),
            pltpu.SemaphoreType.DMA,
            pltpu.SemaphoreType.DMA,
            pltpu.SemaphoreType.DMA,
            pltpu.SemaphoreType.DMA,
        ],
    )


def _sc_seg(wbar, src3):
    return _build_sc_seg()(wbar, src3)


def kernel(x, pos, edge_index, edge_vec,
           W_sc_0, W_lin1_0, W_fc1_0, W_fc2_0, W_lin2_0,
           W_sc_1, W_lin1_1, W_fc1_1, W_fc2_1, W_lin2_1,
           W_sc_2, W_lin1_2, W_fc1_2, W_fc2_2, W_lin2_2):
    src3 = edge_index[0].astype(jnp.int32).reshape(NT, NCH, CH)
    dst3 = edge_index[1].astype(jnp.int32).reshape(NT, NCH, CH)
    evt = edge_vec.T
    wbar0 = _wbar_call(evt, W_fc1_0.T, W_fc2_0.T)
    wbar1 = _wbar_call(evt, W_fc1_1.T, W_fc2_1.T)
    wbar2 = _wbar_call(evt, W_fc1_2.T, W_fc2_2.T)
    s0, xa0, xb0 = _node_in_call(x, W_sc_0, W_lin1_0)
    agg0 = _sc_conv(xa0, xb0, wbar0, src3, dst3)
    s1, xa1, xb1 = _epi_mid_call(s0, agg0, W_lin2_0, W_sc_1, W_lin1_1)
    agg1 = _sc_conv(xa1, xb1, wbar1, src3, dst3)
    h1, xa2, xb2 = _epi_last_call(s1, agg1, W_lin2_1, W_lin1_2)
    s2 = _sc_seg(wbar2, src3)
    return _final_call(h1, xa2, xb2, s2, W_sc_2, W_lin2_2)


# speedup vs baseline: 1.1535x; 1.1052x over previous
import functools

import numpy as np
import jax
import jax.numpy as jnp
from jax import lax
from jax.experimental import pallas as pl
from jax.experimental.pallas import tpu as pltpu
from jax.experimental.pallas import tpu_sc as plsc

N = 10000
E = 320000
D = 128
NB = 10
RN = 64
MAX_R = 3.5
HF = 64
NT = 16
EPT = E // NT
CH = 80
NCH = EPT // CH
RQ = 624
REM = N - NT * RQ
ZR = 156
SIN = float(np.sin(np.pi / 8.0))
COS = float(np.cos(np.pi / 8.0))
INV_SQRT_D = float(1.0 / np.sqrt(D))
INV_SQRT_NB = float(1.0 / np.sqrt(NB))
INV_SQRT_RN = float(1.0 / np.sqrt(RN))
INV_SQRT_NEI = float(1.0 / np.sqrt(32.0))
INV_SQRT_N = float(1.0 / np.sqrt(10000.0))

@functools.cache
def _get_mesh():
    return plsc.VectorSubcoreMesh(core_axis_name="c", subcore_axis_name="s")


_BE = 6400


def _wbar_body(evt, f10, f20, f11, f21, f12, f22, o0, o1, o2):
    v = evt[...]
    el2 = jnp.sum(v * v, axis=0, keepdims=True)
    elen = jnp.sqrt(el2)
    step = MAX_R / (NB - 1)
    centers = lax.broadcasted_iota(jnp.int32, (NB, 1), 0).astype(jnp.float32) * step
    diff = (elen - centers) / step
    eln = jnp.exp(-diff * diff) * (float(np.sqrt(NB)) / 1.12)
    u = 2.0 * (elen / MAX_R - 1.0)
    cut = (1.0 - jnp.cos(jnp.pi * u)) * 0.5
    cut = jnp.where(u > 0.0, 0.0, cut)
    cut = jnp.where(u < -1.0, 1.0, cut)
    for f1, f2, o in ((f10, f20, o0), (f11, f21, o1), (f12, f22, o2)):
        ht = jax.nn.silu(
            jnp.dot(f1[...], eln, preferred_element_type=jnp.float32)
            * INV_SQRT_NB)
        ht = ht * (cut * INV_SQRT_RN)
        wt = jnp.dot(f2[...], ht, preferred_element_type=jnp.float32)
        wb = wt.T
        o[0] = wb[:, :HF]
        o[1] = wb[:, HF:]


def _wbar_call(evt, f10, f20, f11, f21, f12, f22):
    return pl.pallas_call(
        _wbar_body,
        grid=(E // _BE,),
        in_specs=[
            pl.BlockSpec((3, _BE), lambda i: (0, i)),
            pl.BlockSpec((RN, NB), lambda i: (0, 0)),
            pl.BlockSpec((D, RN), lambda i: (0, 0)),
            pl.BlockSpec((RN, NB), lambda i: (0, 0)),
            pl.BlockSpec((D, RN), lambda i: (0, 0)),
            pl.BlockSpec((RN, NB), lambda i: (0, 0)),
            pl.BlockSpec((D, RN), lambda i: (0, 0)),
        ],
        out_specs=[pl.BlockSpec((2, _BE, HF), lambda i: (0, i, 0))] * 3,
        out_shape=[jax.ShapeDtypeStruct((2, E, HF), jnp.float32)] * 3,
    )(evt, f10, f20, f11, f21, f12, f22)


_BN = 2000


def _node_in_body(x_r, wsc_r, wl1_r, s_o, xa_o, xb_o):
    xv = x_r[...]
    s_o[...] = jnp.dot(xv, wsc_r[...], preferred_element_type=jnp.float32) * INV_SQRT_D
    xl = jnp.dot(xv, wl1_r[...], preferred_element_type=jnp.float32) * INV_SQRT_D
    xa_o[...] = xl[:, :HF]
    xb_o[...] = xl[:, HF:]


def _node_in_call(x, wsc, wl1):
    wspec = pl.BlockSpec((D, D), lambda i: (0, 0))
    return pl.pallas_call(
        _node_in_body,
        grid=(N // _BN,),
        in_specs=[pl.BlockSpec((_BN, D), lambda i: (i, 0)), wspec, wspec],
        out_specs=[pl.BlockSpec((_BN, D), lambda i: (i, 0)),
                   pl.BlockSpec((_BN, HF), lambda i: (i, 0)),
                   pl.BlockSpec((_BN, HF), lambda i: (i, 0))],
        out_shape=[jax.ShapeDtypeStruct((N, D), jnp.float32),
                   jax.ShapeDtypeStruct((N, HF), jnp.float32),
                   jax.ShapeDtypeStruct((N, HF), jnp.float32)],
    )(x, wsc, wl1)


def _epi_mid_body(s_r, agg_r, wl2_r, wsc_r, wl1_r, s_o, xa_o, xb_o):
    a = jnp.concatenate([agg_r[0], agg_r[1]], axis=1)
    out = (jnp.dot(a, wl2_r[...], preferred_element_type=jnp.float32)
           * (INV_SQRT_NEI * INV_SQRT_D))
    h = jax.nn.silu(SIN * s_r[...] + COS * out)
    s_o[...] = jnp.dot(h, wsc_r[...], preferred_element_type=jnp.float32) * INV_SQRT_D
    xl = jnp.dot(h, wl1_r[...], preferred_element_type=jnp.float32) * INV_SQRT_D
    xa_o[...] = xl[:, :HF]
    xb_o[...] = xl[:, HF:]


def _epi_mid_call(s, agg, wl2, wsc, wl1):
    wspec = pl.BlockSpec((D, D), lambda i: (0, 0))
    return pl.pallas_call(
        _epi_mid_body,
        grid=(N // _BN,),
        in_specs=[pl.BlockSpec((_BN, D), lambda i: (i, 0)),
                  pl.BlockSpec((2, _BN, HF), lambda i: (0, i, 0)),
                  wspec, wspec, wspec],
        out_specs=[pl.BlockSpec((_BN, D), lambda i: (i, 0)),
                   pl.BlockSpec((_BN, HF), lambda i: (i, 0)),
                   pl.BlockSpec((_BN, HF), lambda i: (i, 0))],
        out_shape=[jax.ShapeDtypeStruct((N, D), jnp.float32),
                   jax.ShapeDtypeStruct((N, HF), jnp.float32),
                   jax.ShapeDtypeStruct((N, HF), jnp.float32)],
    )(s, agg, wl2, wsc, wl1)


def _epi_last_body(s_r, agg_r, wl2_r, wl1_r, h_o, xa_o, xb_o):
    a = jnp.concatenate([agg_r[0], agg_r[1]], axis=1)
    out = (jnp.dot(a, wl2_r[...], preferred_element_type=jnp.float32)
           * (INV_SQRT_NEI * INV_SQRT_D))
    h = jax.nn.silu(SIN * s_r[...] + COS * out)
    h_o[...] = h
    xl = jnp.dot(h, wl1_r[...], preferred_element_type=jnp.float32) * INV_SQRT_D
    xa_o[...] = xl[:, :HF]
    xb_o[...] = xl[:, HF:]


def _epi_last_call(s, agg, wl2, wl1):
    wspec = pl.BlockSpec((D, D), lambda i: (0, 0))
    return pl.pallas_call(
        _epi_last_body,
        grid=(N // _BN,),
        in_specs=[pl.BlockSpec((_BN, D), lambda i: (i, 0)),
                  pl.BlockSpec((2, _BN, HF), lambda i: (0, i, 0)),
                  wspec, wspec],
        out_specs=[pl.BlockSpec((_BN, D), lambda i: (i, 0)),
                   pl.BlockSpec((_BN, HF), lambda i: (i, 0)),
                   pl.BlockSpec((_BN, HF), lambda i: (i, 0))],
        out_shape=[jax.ShapeDtypeStruct((N, D), jnp.float32),
                   jax.ShapeDtypeStruct((N, HF), jnp.float32),
                   jax.ShapeDtypeStruct((N, HF), jnp.float32)],
    )(s, agg, wl2, wl1)


def _final_body(h_r, xa_r, xb_r, s2_r, wsc_r, wl2_r, o):
    hsum = jnp.sum(h_r[...], axis=0, keepdims=True)
    x0 = jnp.sum(xa_r[...] * s2_r[0], axis=0, keepdims=True)
    x1 = jnp.sum(xb_r[...] * s2_r[1], axis=0, keepdims=True)
    xsum = jnp.concatenate([x0, x1], axis=1)
    ssum = jnp.dot(hsum, wsc_r[...], preferred_element_type=jnp.float32) * INV_SQRT_D
    osum = (jnp.dot(xsum, wl2_r[...], preferred_element_type=jnp.float32)
            * (INV_SQRT_NEI * INV_SQRT_D))
    o[...] = (SIN * ssum + COS * osum) * INV_SQRT_N


def _final_call(h1, xa2, xb2, s2, wsc2, wl22):
    return pl.pallas_call(
        _final_body,
        grid=(1,),
        in_specs=[pl.BlockSpec((N, D), lambda i: (0, 0)),
                  pl.BlockSpec((N, HF), lambda i: (0, 0)),
                  pl.BlockSpec((N, HF), lambda i: (0, 0)),
                  pl.BlockSpec((2, N, HF), lambda i: (0, 0, 0)),
                  pl.BlockSpec((D, 1), lambda i: (0, 0)),
                  pl.BlockSpec((D, 1), lambda i: (0, 0))],
        out_specs=pl.BlockSpec((1, 1), lambda i: (0, 0)),
        out_shape=jax.ShapeDtypeStruct((1, 1), jnp.float32),
    )(h1, xa2, xb2, s2, wsc2, wl22)


def _zero_rows(zbuf):
    def zrow(i, _):
        for j in range(HF // 16):
            zbuf[i, pl.ds(j * 16, 16)] = jnp.zeros((16,), jnp.float32)
        return 0
    lax.fori_loop(0, ZR, zrow, 0)


def _sc_conv_body(xa_h, xb_h, wbar_h, src_h, dst_h, agg_o,
                  aggsh, srcv, dstv, rows0, rows1, wch0, wch1, zbuf,
                  sem_g0, sem_g1, sem_w0, sem_w1, sem_s0, sem_s1):
    c = lax.axis_index("c")
    s = lax.axis_index("s")
    _zero_rows(zbuf)
    for b in range(RQ // ZR):
        pltpu.sync_copy(zbuf, aggsh.at[pl.ds(s * RQ + b * ZR, ZR)])

    @pl.when(s == 0)
    def _():
        pltpu.sync_copy(zbuf.at[pl.ds(0, REM)], aggsh.at[pl.ds(NT * RQ, REM)])

    pltpu.sync_copy(src_h.at[s], srcv)
    pltpu.sync_copy(dst_h.at[s], dstv)
    plsc.subcore_barrier()

    def issue(k, rows, sem_g, wch, sem_w):
        @pl.when(c == 0)
        def _():
            pltpu.async_copy(xa_h.at[srcv.at[k]], rows, sem_g)

        @pl.when(c == 1)
        def _():
            pltpu.async_copy(xb_h.at[srcv.at[k]], rows, sem_g)

        pltpu.async_copy(wbar_h.at[c, pl.ds(s * EPT + k * CH, CH)], wch, sem_w)

    def wait_in(k, rows, sem_g, wch, sem_w):
        @pl.when(c == 0)
        def _():
            pltpu.make_async_copy(xa_h.at[srcv.at[k]], rows, sem_g).wait()

        @pl.when(c == 1)
        def _():
            pltpu.make_async_copy(xb_h.at[srcv.at[k]], rows, sem_g).wait()

        pltpu.make_async_copy(
            wbar_h.at[c, pl.ds(s * EPT + k * CH, CH)], wch, sem_w).wait()

    def mul(rows, wch):
        def mrow(i, _):
            for rr in range(4):
                r = i * 4 + rr
                for j in range(HF // 16):
                    sl = pl.ds(j * 16, 16)
                    rows[r, sl] = rows[r, sl] * wch[r, sl]
            return 0
        lax.fori_loop(0, CH // 4, mrow, 0)

    issue(0, rows0, sem_g0, wch0, sem_w0)

    def pair(i, _):
        k0 = 2 * i
        k1 = 2 * i + 1
        issue(k1, rows1, sem_g1, wch1, sem_w1)
        wait_in(k0, rows0, sem_g0, wch0, sem_w0)
        mul(rows0, wch0)
        cs0 = pltpu.async_copy(rows0, aggsh.at[dstv.at[k0]], sem_s0, add=True)
        wait_in(k1, rows1, sem_g1, wch1, sem_w1)
        mul(rows1, wch1)
        cs1 = pltpu.async_copy(rows1, aggsh.at[dstv.at[k1]], sem_s1, add=True)
        cs0.wait()

        @pl.when(i < NCH // 2 - 1)
        def _():
            issue(k0 + 2, rows0, sem_g0, wch0, sem_w0)

        cs1.wait()
        return 0

    lax.fori_loop(0, NCH // 2, pair, 0)
    plsc.subcore_barrier()
    pltpu.sync_copy(aggsh.at[pl.ds(s * RQ, RQ)], agg_o.at[c, pl.ds(s * RQ, RQ)])

    @pl.when(s == 0)
    def _():
        pltpu.sync_copy(aggsh.at[pl.ds(NT * RQ, REM)], agg_o.at[c, pl.ds(NT * RQ, REM)])


@functools.cache
def _build_sc_conv():
    return pl.kernel(
        _sc_conv_body,
        out_type=jax.ShapeDtypeStruct((2, N, HF), jnp.float32),
        mesh=_get_mesh(),
        compiler_params=pltpu.CompilerParams(use_tc_tiling_on_sc=False),
        scratch_types=[
            pltpu.VMEM_SHARED((N, HF), jnp.float32),
            pltpu.VMEM((NCH, CH), jnp.int32),
            pltpu.VMEM((NCH, CH), jnp.int32),
            pltpu.VMEM((CH, HF), jnp.float32),
            pltpu.VMEM((CH, HF), jnp.float32),
            pltpu.VMEM((CH, HF), jnp.float32),
            pltpu.VMEM((CH, HF), jnp.float32),
            pltpu.VMEM((ZR, HF), jnp.float32),
            pltpu.SemaphoreType.DMA,
            pltpu.SemaphoreType.DMA,
            pltpu.SemaphoreType.DMA,
            pltpu.SemaphoreType.DMA,
            pltpu.SemaphoreType.DMA,
            pltpu.SemaphoreType.DMA,
        ],
    )


def _sc_conv(xa, xb, wbar, src3, dst3):
    return _build_sc_conv()(xa, xb, wbar, src3, dst3)


def _sc_seg_body(wbar_h, src_h, s_o, aggsh, srcv, wch0, wch1, zbuf,
                 sem_w0, sem_w1, sem_s0, sem_s1):
    c = lax.axis_index("c")
    s = lax.axis_index("s")
    _zero_rows(zbuf)
    for b in range(RQ // ZR):
        pltpu.sync_copy(zbuf, aggsh.at[pl.ds(s * RQ + b * ZR, ZR)])

    @pl.when(s == 0)
    def _():
        pltpu.sync_copy(zbuf.at[pl.ds(0, REM)], aggsh.at[pl.ds(NT * RQ, REM)])

    pltpu.sync_copy(src_h.at[s], srcv)
    plsc.subcore_barrier()

    def issue(k, wch, sem_w):
        pltpu.async_copy(wbar_h.at[c, pl.ds(s * EPT + k * CH, CH)], wch, sem_w)

    def wait_in(k, wch, sem_w):
        pltpu.make_async_copy(
            wbar_h.at[c, pl.ds(s * EPT + k * CH, CH)], wch, sem_w).wait()

    issue(0, wch0, sem_w0)

    def pair(i, _):
        k0 = 2 * i
        k1 = 2 * i + 1
        issue(k1, wch1, sem_w1)
        wait_in(k0, wch0, sem_w0)
        cs0 = pltpu.async_copy(wch0, aggsh.at[srcv.at[k0]], sem_s0, add=True)
        wait_in(k1, wch1, sem_w1)
        cs1 = pltpu.async_copy(wch1, aggsh.at[srcv.at[k1]], sem_s1, add=True)
        cs0.wait()

        @pl.when(i < NCH // 2 - 1)
        def _():
            issue(k0 + 2, wch0, sem_w0)

        cs1.wait()
        return 0

    lax.fori_loop(0, NCH // 2, pair, 0)
    plsc.subcore_barrier()
    pltpu.sync_copy(aggsh.at[pl.ds(s * RQ, RQ)], s_o.at[c, pl.ds(s * RQ, RQ)])

    @pl.when(s == 0)
    def _():
        pltpu.sync_copy(aggsh.at[pl.ds(NT * RQ, REM)], s_o.at[c, pl.ds(NT * RQ, REM)])


@functools.cache
def _build_sc_seg():
    return pl.kernel(
        _sc_seg_body,
        out_type=jax.ShapeDtypeStruct((2, N, HF), jnp.float32),
        mesh=_get_mesh(),
        compiler_params=pltpu.CompilerParams(use_tc_tiling_on_sc=False),
        scratch_types=[
            pltpu.VMEM_SHARED((N, HF), jnp.float32),
            pltpu.VMEM((NCH, CH), jnp.int32),
            pltpu.VMEM((CH, HF), jnp.float32),
            pltpu.VMEM((CH, HF), jnp.float32),
            pltpu.VMEM((ZR, HF), jnp.float32),
            pltpu.SemaphoreType.DMA,
            pltpu.SemaphoreType.DMA,
            pltpu.SemaphoreType.DMA,
            pltpu.SemaphoreType.DMA,
        ],
    )


def _sc_seg(wbar, src3):
    return _build_sc_seg()(wbar, src3)


def kernel(x, pos, edge_index, edge_vec,
           W_sc_0, W_lin1_0, W_fc1_0, W_fc2_0, W_lin2_0,
           W_sc_1, W_lin1_1, W_fc1_1, W_fc2_1, W_lin2_1,
           W_sc_2, W_lin1_2, W_fc1_2, W_fc2_2, W_lin2_2):
    src3 = edge_index[0].astype(jnp.int32).reshape(NT, NCH, CH)
    dst3 = edge_index[1].astype(jnp.int32).reshape(NT, NCH, CH)
    evt = edge_vec.T
    wbar0, wbar1, wbar2 = _wbar_call(
        evt, W_fc1_0.T, W_fc2_0.T, W_fc1_1.T, W_fc2_1.T, W_fc1_2.T, W_fc2_2.T)
    s0, xa0, xb0 = _node_in_call(x, W_sc_0, W_lin1_0)
    agg0 = _sc_conv(xa0, xb0, wbar0, src3, dst3)
    s1, xa1, xb1 = _epi_mid_call(s0, agg0, W_lin2_0, W_sc_1, W_lin1_1)
    agg1 = _sc_conv(xa1, xb1, wbar1, src3, dst3)
    h1, xa2, xb2 = _epi_last_call(s1, agg1, W_lin2_1, W_lin1_2)
    s2 = _sc_seg(wbar2, src3)
    return _final_call(h1, xa2, xb2, s2, W_sc_2, W_lin2_2)
